# baseline (speedup 1.0000x reference)
import jax
import jax.numpy as jnp
from jax import lax
from jax.experimental import pallas as pl
from jax.experimental.pallas import tpu as pltpu
from jax.experimental.pallas import tpu_sc as plsc

_N_REL = 1000
_N_ENT = 1000000
_DIM = 32
_BATCH = 16384
_GAMMA = 12.0
_L = 16
_NC = 2
_NS = 16
_NW = _NC * _NS
_BPW = _BATCH // _NW
_NCHUNK = _BPW // _L
_TINY = 1e-24
_FLAT = _N_ENT * _DIM


def _rsqrt(s):
    i = plsc.bitcast(s, jnp.int32)
    i = jnp.int32(0x5F3759DF) - lax.shift_right_arithmetic(i, 1)
    y = plsc.bitcast(i, jnp.float32)
    for _ in range(4):
        y = y * (1.5 - 0.5 * s * y * y)
    return y


def _score_body(h_hbm, r_hbm, t_hbm, entf_hbm, pentf_hbm, rel_hbm, prel_hbm,
                out_hbm,
                h_v, t_v, rm_v, sgn_v,
                hv_rows, tv_rows, hp_rows, tp_rows, rv_rows,
                prel_v, out_v, sem, sem2):
    wid = lax.axis_index("s") * _NC + lax.axis_index("c")
    base = wid * _BPW
    iota = lax.broadcasted_iota(jnp.int32, (_L,), 0)

    pltpu.sync_copy(h_hbm.at[pl.ds(base, _BPW)], h_v)
    pltpu.sync_copy(t_hbm.at[pl.ds(base, _BPW)], t_v)
    pltpu.sync_copy(r_hbm.at[pl.ds(base, _BPW)], rm_v)
    prel_cp = pltpu.make_async_copy(prel_hbm, prel_v, sem2)
    prel_cp.start()

    def _prep_chunk(c, carry):
        idx = c * _L + iota
        rr = plsc.load_gather(rm_v, [idx])
        plsc.store_scatter(rm_v, [idx], lax.rem(rr, jnp.int32(_N_REL)))
        sgn = jnp.where(rr < _N_REL, jnp.float32(1.0), jnp.float32(-1.0))
        plsc.store_scatter(sgn_v, [idx], sgn)
        return carry

    lax.fori_loop(0, _NCHUNK, _prep_chunk, 0)

    def _fires(d):
        esl = entf_hbm.at[d]
        psl = pentf_hbm.at[d]
        return (
            pltpu.make_async_copy(esl.at[h_v], hv_rows.at[d], sem),
            pltpu.make_async_copy(esl.at[t_v], tv_rows.at[d], sem),
            pltpu.make_async_copy(psl.at[h_v], hp_rows.at[d], sem),
            pltpu.make_async_copy(psl.at[t_v], tp_rows.at[d], sem),
            pltpu.make_async_copy(rel_hbm.at[d].at[rm_v], rv_rows.at[d], sem),
        )

    for cp in _fires(0):
        cp.start()

    def _pipe(d, carry):
        for cp in _fires(d):
            cp.start()
        for cp in _fires(d - 1):
            cp.wait()
        return carry

    lax.fori_loop(1, _DIM, _pipe, 0)
    for cp in _fires(_DIM - 1):
        cp.wait()
    prel_cp.wait()

    def _chunk(c, carry):
        eb = pl.multiple_of(c * _L, _L)
        rmc = rm_v[pl.ds(eb, _L)]
        zero = jnp.zeros((_L,), jnp.float32)
        shh = stt = srr = spp = sht = shr = shp = str_ = stp = srp = dh = dt = zero
        for d in range(_DIM):
            dcol = jnp.full((_L,), d, jnp.int32)
            hd = hv_rows[d, pl.ds(eb, _L)]
            td = tv_rows[d, pl.ds(eb, _L)]
            hpd = hp_rows[d, pl.ds(eb, _L)]
            tpd = tp_rows[d, pl.ds(eb, _L)]
            rd = rv_rows[d, pl.ds(eb, _L)]
            pd = plsc.load_gather(prel_v, [dcol, rmc])
            shh += hd * hd
            stt += td * td
            srr += rd * rd
            spp += pd * pd
            sht += hd * td
            shr += hd * rd
            shp += hd * pd
            str_ += td * rd
            stp += td * pd
            srp += rd * pd
            dh += hd * hpd
            dt += td * tpd

        a = _rsqrt(jnp.maximum(shh, _TINY))
        cc = _rsqrt(jnp.maximum(stt, _TINY))
        rin = _rsqrt(jnp.maximum(srr, _TINY))
        bh = a * dh
        bt = cc * dt
        yh = a * a * shh + 2.0 * a * bh * shp + bh * bh * spp
        yt = cc * cc * stt + 2.0 * cc * bt * stp + bt * bt * spp
        iyh = _rsqrt(jnp.maximum(yh, _TINY))
        iyt = _rsqrt(jnp.maximum(yt, _TINY))
        sgn = sgn_v[pl.ds(eb, _L)]
        ch = iyh * a
        ct = -(iyt * cc)
        cr = sgn * rin
        cp_ = iyh * bh - iyt * bt
        s = (ch * ch * shh + ct * ct * stt + cr * cr * srr + cp_ * cp_ * spp
             + 2.0 * (ch * ct * sht + ch * cr * shr + ch * cp_ * shp
                      + ct * cr * str_ + ct * cp_ * stp + cr * cp_ * srp))
        s = jnp.maximum(s, 0.0)
        dist = s * _rsqrt(jnp.maximum(s, _TINY))
        out_v[pl.ds(eb, _L)] = _GAMMA - dist
        return carry

    lax.fori_loop(0, _NCHUNK, _chunk, 0)

    pltpu.sync_copy(out_v, out_hbm.at[pl.ds(base, _BPW)])


_score = pl.kernel(
    _score_body,
    out_type=jax.ShapeDtypeStruct((_BATCH,), jnp.float32),
    mesh=plsc.VectorSubcoreMesh(core_axis_name="c", subcore_axis_name="s"),
    compiler_params=pltpu.CompilerParams(
        needs_layout_passes=False, use_tc_tiling_on_sc=False),
    scratch_types=[
        pltpu.VMEM((_BPW,), jnp.int32),
        pltpu.VMEM((_BPW,), jnp.int32),
        pltpu.VMEM((_BPW,), jnp.int32),
        pltpu.VMEM((_BPW,), jnp.float32),
        pltpu.VMEM((_DIM, _BPW), jnp.float32),
        pltpu.VMEM((_DIM, _BPW), jnp.float32),
        pltpu.VMEM((_DIM, _BPW), jnp.float32),
        pltpu.VMEM((_DIM, _BPW), jnp.float32),
        pltpu.VMEM((_DIM, _BPW), jnp.float32),
        pltpu.VMEM((_DIM, _N_REL), jnp.float32),
        pltpu.VMEM((_BPW,), jnp.float32),
        pltpu.SemaphoreType.DMA,
        pltpu.SemaphoreType.DMA,
    ],
)


def kernel(h, r, t, ent_embed, rel_embed, proj_ent_embed, proj_rel_embed):
    h = jnp.asarray(h, jnp.int32)
    r = jnp.asarray(r, jnp.int32)
    t = jnp.asarray(t, jnp.int32)
    return _score(h, r, t, ent_embed.T, proj_ent_embed.T,
                  rel_embed.T, proj_rel_embed.T)
